# baseline (speedup 1.0000x reference)
import jax
import jax.numpy as jnp
from jax.experimental import pallas as pl
from jax.experimental.pallas import tpu as pltpu

NUM_ROOT = 64
NUM_TRUNK = 10000
D = 128
IN_DEG = 2
BLK = 200
NUM_BLOCKS = NUM_TRUNK // BLK


def _dag_kernel(idx_ref, root_ref, w1a_ref, w1b_ref, b1_ref, w2_ref, b2_ref,
                out_ref, x0_ref, x1_ref, dep_ref, sem):
    b = pl.program_id(0)
    s = b * BLK

    @pl.when(b == 0)
    def _init():
        out_ref[0:NUM_ROOT, :] = root_ref[...]
        out_ref[NUM_ROOT:, :] = jnp.zeros((NUM_TRUNK, D), jnp.float32)

    base = s + NUM_ROOT

    def mlp():
        h = (jnp.dot(x0_ref[...], w1a_ref[...], preferred_element_type=jnp.float32)
             + jnp.dot(x1_ref[...], w1b_ref[...], preferred_element_type=jnp.float32))
        h = jax.nn.gelu(h + b1_ref[...])
        new = jnp.dot(h, w2_ref[...], preferred_element_type=jnp.float32)
        return new + b2_ref[...]

    m = jnp.int32(0)
    for j in range(BLK):
        i0 = idx_ref[2 * (s + j)]
        i1 = idx_ref[2 * (s + j) + 1]
        x0_ref[pl.ds(j, 1), :] = out_ref[pl.ds(i0, 1), :]
        x1_ref[pl.ds(j, 1), :] = out_ref[pl.ds(i1, 1), :]
        dep_ref[m] = j
        inb = jnp.logical_or(i0 >= base, i1 >= base)
        m = m + inb.astype(jnp.int32)

    out_ref[pl.ds(base, BLK), :] = mlp()

    def refresh(carry):
        p, _ = carry

        def start_body(k, c):
            j = dep_ref[k]
            i0 = idx_ref[2 * (s + j)]
            i1 = idx_ref[2 * (s + j) + 1]
            pltpu.make_async_copy(out_ref.at[pl.ds(i0, 1), :],
                                  x0_ref.at[pl.ds(j, 1), :], sem).start()
            pltpu.make_async_copy(out_ref.at[pl.ds(i1, 1), :],
                                  x1_ref.at[pl.ds(j, 1), :], sem).start()
            return c

        jax.lax.fori_loop(0, m, start_body, 0)

        def wait_body(k, c):
            pltpu.make_async_copy(out_ref.at[pl.ds(0, 1), :],
                                  x0_ref.at[pl.ds(0, 1), :], sem).wait()
            pltpu.make_async_copy(out_ref.at[pl.ds(0, 1), :],
                                  x1_ref.at[pl.ds(0, 1), :], sem).wait()
            return c

        jax.lax.fori_loop(0, m, wait_body, 0)

        new = mlp()
        old = out_ref[pl.ds(base, BLK), :]
        nchanged = jnp.sum((new != old).astype(jnp.float32))
        out_ref[pl.ds(base, BLK), :] = new
        return (p + 1, nchanged > 0.0)

    def cond(carry):
        p, go = carry
        return jnp.logical_and(go, p < BLK)

    jax.lax.while_loop(cond, refresh, (jnp.int32(1), m > 0))


@jax.jit
def kernel(root_node_embeddings, trunk_node_inputs_indices, trunk_node_types,
           W1, b1, W2, b2):
    del trunk_node_types
    grid_spec = pltpu.PrefetchScalarGridSpec(
        num_scalar_prefetch=1,
        grid=(NUM_BLOCKS,),
        in_specs=[
            pl.BlockSpec((NUM_ROOT, D), lambda b, idx: (0, 0)),
            pl.BlockSpec((D, 2 * D), lambda b, idx: (0, 0)),
            pl.BlockSpec((D, 2 * D), lambda b, idx: (0, 0)),
            pl.BlockSpec((1, 2 * D), lambda b, idx: (0, 0)),
            pl.BlockSpec((2 * D, D), lambda b, idx: (0, 0)),
            pl.BlockSpec((1, D), lambda b, idx: (0, 0)),
        ],
        out_specs=pl.BlockSpec((NUM_ROOT + NUM_TRUNK, D), lambda b, idx: (0, 0)),
        scratch_shapes=[
            pltpu.VMEM((BLK, D), jnp.float32),
            pltpu.VMEM((BLK, D), jnp.float32),
            pltpu.SMEM((BLK,), jnp.int32),
            pltpu.SemaphoreType.DMA,
        ],
    )
    out = pl.pallas_call(
        _dag_kernel,
        grid_spec=grid_spec,
        out_shape=jax.ShapeDtypeStruct((NUM_ROOT + NUM_TRUNK, D), jnp.float32),
        compiler_params=pltpu.CompilerParams(
            dimension_semantics=("arbitrary",),
        ),
    )(trunk_node_inputs_indices.reshape(-1),
      root_node_embeddings,
      W1[:D], W1[D:], b1.reshape(1, 2 * D), W2, b2.reshape(1, D))
    return out


# speedup vs baseline: 1.3500x; 1.3500x over previous
import jax
import jax.numpy as jnp
from jax.experimental import pallas as pl
from jax.experimental.pallas import tpu as pltpu

NUM_ROOT = 64
NUM_TRUNK = 10000
D = 128
IN_DEG = 2
BLK = 200
NUM_BLOCKS = NUM_TRUNK // BLK


def _dag_kernel(idx_ref, root_ref, w1a_ref, w1b_ref, b1_ref, w2_ref, b2_ref,
                out_ref, x0_ref, x1_ref):
    b = pl.program_id(0)
    s = b * BLK

    @pl.when(b == 0)
    def _init():
        out_ref[0:NUM_ROOT, :] = root_ref[...]
        out_ref[NUM_ROOT:, :] = jnp.zeros((NUM_TRUNK, D), jnp.float32)

    base = s + NUM_ROOT

    s2 = 2 * s

    def pass_body(carry):
        p, _ = carry
        for j in range(BLK):
            i0 = idx_ref[s2 + 2 * j]
            i1 = idx_ref[s2 + 2 * j + 1]
            x0_ref[pl.ds(j, 1), :] = out_ref[pl.ds(i0, 1), :]
            x1_ref[pl.ds(j, 1), :] = out_ref[pl.ds(i1, 1), :]

        h = (jnp.dot(x0_ref[...], w1a_ref[...], preferred_element_type=jnp.float32)
             + jnp.dot(x1_ref[...], w1b_ref[...], preferred_element_type=jnp.float32))
        h = jax.nn.gelu(h + b1_ref[...])
        new = jnp.dot(h, w2_ref[...], preferred_element_type=jnp.float32)
        new = new + b2_ref[...]
        old = out_ref[pl.ds(base, BLK), :]
        nchanged = jnp.sum((new != old).astype(jnp.float32))
        out_ref[pl.ds(base, BLK), :] = new
        return (p + 1, nchanged > 0.0)

    def cond(carry):
        p, go = carry
        return jnp.logical_and(go, p < BLK)

    jax.lax.while_loop(cond, pass_body, (jnp.int32(0), jnp.bool_(True)))


@jax.jit
def kernel(root_node_embeddings, trunk_node_inputs_indices, trunk_node_types,
           W1, b1, W2, b2):
    del trunk_node_types
    grid_spec = pltpu.PrefetchScalarGridSpec(
        num_scalar_prefetch=1,
        grid=(NUM_BLOCKS,),
        in_specs=[
            pl.BlockSpec((NUM_ROOT, D), lambda b, idx: (0, 0)),
            pl.BlockSpec((D, 2 * D), lambda b, idx: (0, 0)),
            pl.BlockSpec((D, 2 * D), lambda b, idx: (0, 0)),
            pl.BlockSpec((1, 2 * D), lambda b, idx: (0, 0)),
            pl.BlockSpec((2 * D, D), lambda b, idx: (0, 0)),
            pl.BlockSpec((1, D), lambda b, idx: (0, 0)),
        ],
        out_specs=pl.BlockSpec((NUM_ROOT + NUM_TRUNK, D), lambda b, idx: (0, 0)),
        scratch_shapes=[
            pltpu.VMEM((BLK, D), jnp.float32),
            pltpu.VMEM((BLK, D), jnp.float32),
        ],
    )
    out = pl.pallas_call(
        _dag_kernel,
        grid_spec=grid_spec,
        out_shape=jax.ShapeDtypeStruct((NUM_ROOT + NUM_TRUNK, D), jnp.float32),
        compiler_params=pltpu.CompilerParams(
            dimension_semantics=("arbitrary",),
        ),
    )(trunk_node_inputs_indices.reshape(-1),
      root_node_embeddings,
      W1[:D], W1[D:], b1.reshape(1, 2 * D), W2, b2.reshape(1, D))
    return out


# speedup vs baseline: 1.4402x; 1.0668x over previous
import jax
import jax.numpy as jnp
from jax.experimental import pallas as pl
from jax.experimental.pallas import tpu as pltpu

NUM_ROOT = 64
NUM_TRUNK = 10000
D = 128
IN_DEG = 2
BLK = 400
NUM_BLOCKS = NUM_TRUNK // BLK


def _dag_kernel(idx_ref, root_ref, w1a_ref, w1b_ref, b1_ref, w2_ref, b2_ref,
                out_ref, x0_ref, x1_ref):
    b = pl.program_id(0)
    s = b * BLK

    @pl.when(b == 0)
    def _init():
        out_ref[0:NUM_ROOT, :] = root_ref[...]
        out_ref[NUM_ROOT:, :] = jnp.zeros((NUM_TRUNK, D), jnp.float32)

    base = s + NUM_ROOT

    s2 = 2 * s

    def pass_body(carry):
        p, _ = carry
        for j in range(BLK):
            i0 = idx_ref[s2 + 2 * j]
            i1 = idx_ref[s2 + 2 * j + 1]
            x0_ref[pl.ds(j, 1), :] = out_ref[pl.ds(i0, 1), :]
            x1_ref[pl.ds(j, 1), :] = out_ref[pl.ds(i1, 1), :]

        h = (jnp.dot(x0_ref[...], w1a_ref[...], preferred_element_type=jnp.float32)
             + jnp.dot(x1_ref[...], w1b_ref[...], preferred_element_type=jnp.float32))
        h = jax.nn.gelu(h + b1_ref[...])
        new = jnp.dot(h, w2_ref[...], preferred_element_type=jnp.float32)
        new = new + b2_ref[...]
        old = out_ref[pl.ds(base, BLK), :]
        nchanged = jnp.sum((new != old).astype(jnp.float32))
        out_ref[pl.ds(base, BLK), :] = new
        return (p + 1, nchanged > 0.0)

    def cond(carry):
        p, go = carry
        return jnp.logical_and(go, p < BLK)

    jax.lax.while_loop(cond, pass_body, (jnp.int32(0), jnp.bool_(True)))


@jax.jit
def kernel(root_node_embeddings, trunk_node_inputs_indices, trunk_node_types,
           W1, b1, W2, b2):
    del trunk_node_types
    grid_spec = pltpu.PrefetchScalarGridSpec(
        num_scalar_prefetch=1,
        grid=(NUM_BLOCKS,),
        in_specs=[
            pl.BlockSpec((NUM_ROOT, D), lambda b, idx: (0, 0)),
            pl.BlockSpec((D, 2 * D), lambda b, idx: (0, 0)),
            pl.BlockSpec((D, 2 * D), lambda b, idx: (0, 0)),
            pl.BlockSpec((1, 2 * D), lambda b, idx: (0, 0)),
            pl.BlockSpec((2 * D, D), lambda b, idx: (0, 0)),
            pl.BlockSpec((1, D), lambda b, idx: (0, 0)),
        ],
        out_specs=pl.BlockSpec((NUM_ROOT + NUM_TRUNK, D), lambda b, idx: (0, 0)),
        scratch_shapes=[
            pltpu.VMEM((BLK, D), jnp.float32),
            pltpu.VMEM((BLK, D), jnp.float32),
        ],
    )
    out = pl.pallas_call(
        _dag_kernel,
        grid_spec=grid_spec,
        out_shape=jax.ShapeDtypeStruct((NUM_ROOT + NUM_TRUNK, D), jnp.float32),
        compiler_params=pltpu.CompilerParams(
            dimension_semantics=("arbitrary",),
        ),
    )(trunk_node_inputs_indices.reshape(-1),
      root_node_embeddings,
      W1[:D], W1[D:], b1.reshape(1, 2 * D), W2, b2.reshape(1, D))
    return out


# speedup vs baseline: 1.5139x; 1.0512x over previous
import jax
import jax.numpy as jnp
from jax.experimental import pallas as pl
from jax.experimental.pallas import tpu as pltpu

NUM_ROOT = 64
NUM_TRUNK = 10000
D = 128
IN_DEG = 2
BLK = 400
NUM_BLOCKS = NUM_TRUNK // BLK
NACC = 8


def _dag_kernel(idx_ref, root_ref, w1a_ref, w1b_ref, b1_ref, w2_ref, b2_ref,
                out_ref, x0_ref, x1_ref):
    b = pl.program_id(0)
    s = b * BLK

    @pl.when(b == 0)
    def _init():
        out_ref[0:NUM_ROOT, :] = root_ref[...]
        out_ref[NUM_ROOT:, :] = jnp.zeros((NUM_TRUNK, D), jnp.float32)

    base = s + NUM_ROOT
    s2 = 2 * s

    def mlp_store():
        h = (jnp.dot(x0_ref[...], w1a_ref[...], preferred_element_type=jnp.float32)
             + jnp.dot(x1_ref[...], w1b_ref[...], preferred_element_type=jnp.float32))
        h = jax.nn.gelu(h + b1_ref[...])
        new = jnp.dot(h, w2_ref[...], preferred_element_type=jnp.float32)
        new = new + b2_ref[...]
        out_ref[pl.ds(base, BLK), :] = new

    for j in range(BLK):
        i0 = idx_ref[s2 + 2 * j]
        i1 = idx_ref[s2 + 2 * j + 1]
        x0_ref[pl.ds(j, 1), :] = out_ref[pl.ds(i0, 1), :]
        x1_ref[pl.ds(j, 1), :] = out_ref[pl.ds(i1, 1), :]
    mlp_store()

    def pass_body(carry):
        p, _ = carry
        accs = [jnp.zeros((1, D), jnp.int32) for _ in range(NACC)]
        for j in range(BLK):
            i0 = idx_ref[s2 + 2 * j]
            i1 = idx_ref[s2 + 2 * j + 1]
            r0 = out_ref[pl.ds(i0, 1), :]
            r1 = out_ref[pl.ds(i1, 1), :]
            ne = ((r0 != x0_ref[pl.ds(j, 1), :]).astype(jnp.int32)
                  | (r1 != x1_ref[pl.ds(j, 1), :]).astype(jnp.int32))
            accs[j % NACC] = accs[j % NACC] | ne
            x0_ref[pl.ds(j, 1), :] = r0
            x1_ref[pl.ds(j, 1), :] = r1
        acc = accs[0]
        for a in accs[1:]:
            acc = acc | a
        changed = jnp.sum(acc) > 0

        @pl.when(changed)
        def _():
            mlp_store()

        return (p + 1, changed)

    def cond(carry):
        p, go = carry
        return jnp.logical_and(go, p < BLK)

    jax.lax.while_loop(cond, pass_body, (jnp.int32(1), jnp.bool_(True)))


@jax.jit
def kernel(root_node_embeddings, trunk_node_inputs_indices, trunk_node_types,
           W1, b1, W2, b2):
    del trunk_node_types
    grid_spec = pltpu.PrefetchScalarGridSpec(
        num_scalar_prefetch=1,
        grid=(NUM_BLOCKS,),
        in_specs=[
            pl.BlockSpec((NUM_ROOT, D), lambda b, idx: (0, 0)),
            pl.BlockSpec((D, 2 * D), lambda b, idx: (0, 0)),
            pl.BlockSpec((D, 2 * D), lambda b, idx: (0, 0)),
            pl.BlockSpec((1, 2 * D), lambda b, idx: (0, 0)),
            pl.BlockSpec((2 * D, D), lambda b, idx: (0, 0)),
            pl.BlockSpec((1, D), lambda b, idx: (0, 0)),
        ],
        out_specs=pl.BlockSpec((NUM_ROOT + NUM_TRUNK, D), lambda b, idx: (0, 0)),
        scratch_shapes=[
            pltpu.VMEM((BLK, D), jnp.float32),
            pltpu.VMEM((BLK, D), jnp.float32),
        ],
    )
    out = pl.pallas_call(
        _dag_kernel,
        grid_spec=grid_spec,
        out_shape=jax.ShapeDtypeStruct((NUM_ROOT + NUM_TRUNK, D), jnp.float32),
        compiler_params=pltpu.CompilerParams(
            dimension_semantics=("arbitrary",),
        ),
    )(trunk_node_inputs_indices.reshape(-1),
      root_node_embeddings,
      W1[:D], W1[D:], b1.reshape(1, 2 * D), W2, b2.reshape(1, D))
    return out


# speedup vs baseline: 1.5145x; 1.0004x over previous
import jax
import jax.numpy as jnp
from jax.experimental import pallas as pl
from jax.experimental.pallas import tpu as pltpu

NUM_ROOT = 64
NUM_TRUNK = 10000
D = 128
IN_DEG = 2
BLK = 200
NUM_BLOCKS = NUM_TRUNK // BLK
NACC = 8


def _dag_kernel(idx_ref, root_ref, w1a_ref, w1b_ref, b1_ref, w2_ref, b2_ref,
                out_ref, x0_ref, x1_ref):
    b = pl.program_id(0)
    s = b * BLK

    @pl.when(b == 0)
    def _init():
        out_ref[0:NUM_ROOT, :] = root_ref[...]
        out_ref[NUM_ROOT:, :] = jnp.zeros((NUM_TRUNK, D), jnp.float32)

    base = s + NUM_ROOT
    s2 = 2 * s

    def mlp_store():
        h = (jnp.dot(x0_ref[...], w1a_ref[...], preferred_element_type=jnp.float32)
             + jnp.dot(x1_ref[...], w1b_ref[...], preferred_element_type=jnp.float32))
        h = jax.nn.gelu(h + b1_ref[...])
        new = jnp.dot(h, w2_ref[...], preferred_element_type=jnp.float32)
        new = new + b2_ref[...]
        out_ref[pl.ds(base, BLK), :] = new

    for j in range(BLK):
        i0 = idx_ref[s2 + 2 * j]
        i1 = idx_ref[s2 + 2 * j + 1]
        x0_ref[pl.ds(j, 1), :] = out_ref[pl.ds(i0, 1), :]
        x1_ref[pl.ds(j, 1), :] = out_ref[pl.ds(i1, 1), :]
    mlp_store()

    def pass_body(carry):
        p, _ = carry
        accs = [jnp.zeros((1, D), jnp.int32) for _ in range(NACC)]
        for j in range(BLK):
            i0 = idx_ref[s2 + 2 * j]
            i1 = idx_ref[s2 + 2 * j + 1]
            r0 = out_ref[pl.ds(i0, 1), :]
            r1 = out_ref[pl.ds(i1, 1), :]
            ne = ((r0 != x0_ref[pl.ds(j, 1), :]).astype(jnp.int32)
                  | (r1 != x1_ref[pl.ds(j, 1), :]).astype(jnp.int32))
            accs[j % NACC] = accs[j % NACC] | ne
            x0_ref[pl.ds(j, 1), :] = r0
            x1_ref[pl.ds(j, 1), :] = r1
        acc = accs[0]
        for a in accs[1:]:
            acc = acc | a
        changed = jnp.sum(acc) > 0

        @pl.when(changed)
        def _():
            mlp_store()

        return (p + 1, changed)

    def cond(carry):
        p, go = carry
        return jnp.logical_and(go, p < BLK)

    jax.lax.while_loop(cond, pass_body, (jnp.int32(1), jnp.bool_(True)))


@jax.jit
def kernel(root_node_embeddings, trunk_node_inputs_indices, trunk_node_types,
           W1, b1, W2, b2):
    del trunk_node_types
    grid_spec = pltpu.PrefetchScalarGridSpec(
        num_scalar_prefetch=1,
        grid=(NUM_BLOCKS,),
        in_specs=[
            pl.BlockSpec((NUM_ROOT, D), lambda b, idx: (0, 0)),
            pl.BlockSpec((D, 2 * D), lambda b, idx: (0, 0)),
            pl.BlockSpec((D, 2 * D), lambda b, idx: (0, 0)),
            pl.BlockSpec((1, 2 * D), lambda b, idx: (0, 0)),
            pl.BlockSpec((2 * D, D), lambda b, idx: (0, 0)),
            pl.BlockSpec((1, D), lambda b, idx: (0, 0)),
        ],
        out_specs=pl.BlockSpec((NUM_ROOT + NUM_TRUNK, D), lambda b, idx: (0, 0)),
        scratch_shapes=[
            pltpu.VMEM((BLK, D), jnp.float32),
            pltpu.VMEM((BLK, D), jnp.float32),
        ],
    )
    out = pl.pallas_call(
        _dag_kernel,
        grid_spec=grid_spec,
        out_shape=jax.ShapeDtypeStruct((NUM_ROOT + NUM_TRUNK, D), jnp.float32),
        compiler_params=pltpu.CompilerParams(
            dimension_semantics=("arbitrary",),
        ),
    )(trunk_node_inputs_indices.reshape(-1),
      root_node_embeddings,
      W1[:D], W1[D:], b1.reshape(1, 2 * D), W2, b2.reshape(1, D))
    return out
